# baseline (speedup 1.0000x reference)
import functools
import math

import jax
import jax.numpy as jnp
from jax import lax
from jax.experimental import pallas as pl
from jax.experimental.pallas import tpu as pltpu
from jax.experimental.pallas import tpu_sc as plsc

SL = (80, 40, 20, 10)
BAL = (8.0, 4.0, 1.0, 0.4)
ANCH = (
    ((4.0, 4.0), (8.0, 7.0), (13.0, 11.0)),
    ((19.0, 17.0), (28.0, 24.0), (41.0, 35.0)),
    ((59.0, 52.0), (87.0, 74.0), (122.0, 98.0)),
    ((165.0, 141.0), (234.0, 187.0), (340.0, 280.0)),
)
IMG = 640.0
BSZ = 128
NT = 2000
NTP = 2048
NROWS = 16
NPAIR = NROWS * NTP
NW = 32
CHUNK = (3 * NTP) // NW
N_CELL = tuple(BSZ * 3 * s * s for s in SL)
WPAD = 32768
EPS = 1e-7

_row_l = [r // 4 for r in range(NROWS)]
_row_a = [r % 4 for r in range(NROWS)]
_ROWV = [a < 3 for a in _row_a]
_S = [float(SL[l]) for l in _row_l]
_AW = [ANCH[l][a][0] * SL[l] / IMG if a < 3 else 1.0
       for l, a in zip(_row_l, _row_a)]
_AH = [ANCH[l][a][1] * SL[l] / IMG if a < 3 else 1.0
       for l, a in zip(_row_l, _row_a)]
_AC = [a if a < 3 else 0 for a in _row_a]
_NC = [N_CELL[l] for l in _row_l]


def _row_consts():
    riota = lax.broadcasted_iota(jnp.int32, (NROWS, 1), 0)
    lrow = riota // 4
    arow = riota % 4
    rowv = arow < 3

    def chain(fn, init):
        out = jnp.full((NROWS, 1), init, jnp.float32)
        for l in range(4):
            for a in range(3):
                out = jnp.where((lrow == l) & (arow == a),
                                jnp.float32(fn(l, a)), out)
        return out

    S = chain(lambda l, a: float(SL[l]), 1.0)
    aw = chain(lambda l, a: ANCH[l][a][0] * SL[l] / IMG, 1.0)
    ah = chain(lambda l, a: ANCH[l][a][1] * SL[l] / IMG, 1.0)
    nc = chain(lambda l, a: float(N_CELL[l]), 0.0)
    ac = jnp.where(rowv, arow, 0)
    return dict(S=S, Si=S.astype(jnp.int32), aw=aw, ah=ah,
                nc=nc.astype(jnp.int32), ac=ac, rowv=rowv, lrow=lrow,
                arow=arow)


def _match(tt_ref):
    rc = _row_consts()
    S, Si, aw, ah, ac, rowv = (rc["S"], rc["Si"], rc["aw"], rc["ah"],
                               rc["ac"], rc["rowv"])

    b = tt_ref[0:1, :]
    x = tt_ref[1:2, :]
    y = tt_ref[2:3, :]
    w = tt_ref[3:4, :]
    h = tt_ref[4:5, :]
    gx = x * S
    gy = y * S
    gw = w * S
    gh = h * S
    rw = gw / aw
    rh = gh / ah
    m = jnp.maximum(jnp.maximum(rw, 1.0 / rw), jnp.maximum(rh, 1.0 / rh))
    valid = (m < 3.0) & rowv
    gi = jnp.clip(gx.astype(jnp.int32), 0, Si - 1)
    gj = jnp.clip(gy.astype(jnp.int32), 0, Si - 1)
    bi = b.astype(jnp.int32)
    cell = ((bi * 3 + ac) * Si + gj) * Si + gi
    return dict(valid=valid, cell=cell, gi=gi, gj=gj, bi=bi, ac=ac, Si=Si,
                gx=gx, gy=gy, gw=gw, gh=gh, aw=aw, ah=ah, nc=rc["nc"],
                lrow=rc["lrow"], arow=rc["arow"])


def _prep_body(tt_ref, idx_ref):
    mt = _match(tt_ref)
    valid, cell = mt["valid"], mt["cell"]
    bb0 = ((mt["bi"] * 12 + mt["ac"] * 4) * mt["Si"] + mt["gj"]) * mt["Si"] + mt["gi"]
    uid = (lax.broadcasted_iota(jnp.int32, (NROWS, NTP), 0) * NTP
           + lax.broadcasted_iota(jnp.int32, (NROWS, NTP), 1))
    spread = (uid * 977) & (WPAD - 1)
    idx_ref[0] = jnp.where(valid, cell, mt["nc"] + spread)
    idx_ref[1] = jnp.where(valid, cell, spread)
    idx_ref[2] = jnp.where(valid, bb0, spread)


def _prep(tt):
    return pl.pallas_call(
        _prep_body,
        grid=(1,),
        in_specs=[pl.BlockSpec((5, NTP), lambda i: (0, 0))],
        out_specs=pl.BlockSpec((3, NROWS, NTP), lambda i: (0, 0, 0)),
        out_shape=jax.ShapeDtypeStruct((3, NROWS, NTP), jnp.int32),
    )(tt)


_MESH = plsc.VectorSubcoreMesh(core_axis_name="c", subcore_axis_name="s")


def _scatter(cells_f):
    @functools.partial(
        pl.kernel,
        out_type=[jax.ShapeDtypeStruct((N_CELL[l] + WPAD,), jnp.int32)
                  for l in range(4)],
        mesh=_MESH,
        scratch_types=([pltpu.VMEM((CHUNK,), jnp.int32)] * 4
                       + [pltpu.VMEM((CHUNK,), jnp.int32)] * 4
                       + [pltpu.SemaphoreType.DMA]),
    )
    def k(idx3_hbm, w0, w1, w2, w3, *scr):
        writers = (w0, w1, w2, w3)
        idx_v = scr[0:4]
        val_v = scr[4:8]
        sem = scr[8]
        wid = lax.axis_index("s") * 2 + lax.axis_index("c")
        bases = [l * (4 * NTP) + wid * CHUNK for l in range(4)]
        loads = [pltpu.make_async_copy(
            idx3_hbm.at[pl.ds(bases[l], CHUNK)], idx_v[l], sem)
            for l in range(4)]
        for cp in loads:
            cp.start()
        for l in range(4):
            for j in range(CHUNK // 16):
                val_v[l][pl.ds(j * 16, 16)] = (
                    lax.iota(jnp.int32, 16) + (bases[l] + j * 16))
        for cp in loads:
            cp.wait()
        scats = [pltpu.make_async_copy(
            val_v[l], writers[l].at[idx_v[l]], sem)
            for l in range(4)]
        for cp in scats:
            cp.start()
        for cp in scats:
            cp.wait()

    return k(cells_f)


def _gather(idx3_f, writers, bb_flats, ob_flats):
    outs = [jax.ShapeDtypeStruct((NPAIR,), jnp.int32),
            jax.ShapeDtypeStruct((5 * NPAIR,), jnp.float32)]

    @functools.partial(
        pl.kernel,
        out_type=outs,
        mesh=_MESH,
        scratch_types=([pltpu.VMEM((CHUNK,), jnp.int32)] * 4
                       + [pltpu.VMEM((CHUNK,), jnp.int32)] * 4
                       + [pltpu.VMEM((4 * CHUNK,), jnp.int32)] * 4
                       + [pltpu.VMEM((CHUNK,), jnp.int32)] * 4
                       + [pltpu.VMEM((CHUNK,), jnp.float32)] * 4
                       + [pltpu.VMEM((4 * CHUNK,), jnp.float32)] * 4
                       + [pltpu.SemaphoreType.DMA]),
    )
    def k(idx3_hbm, w0, w1, w2, w3,
          bf0, bf1, bf2, bf3, of0, of1, of2, of3,
          wb_o, pf_o, *scr):
        wr = (w0, w1, w2, w3)
        bf = (bf0, bf1, bf2, bf3)
        of = (of0, of1, of2, of3)
        ic_v = scr[0:4]
        ig_v = scr[4:8]
        ib_v = scr[8:12]
        vi_v = scr[12:16]
        vf_v = scr[16:20]
        vb_v = scr[20:24]
        sem = scr[24]
        wid = lax.axis_index("s") * 2 + lax.axis_index("c")
        bases = [l * (4 * NTP) + wid * CHUNK for l in range(4)]
        sls = [pl.ds(bases[l], CHUNK) for l in range(4)]
        loads = []
        for l in range(4):
            loads.append(pltpu.make_async_copy(
                idx3_hbm.at[pl.ds(bases[l], CHUNK)], ic_v[l], sem))
            loads.append(pltpu.make_async_copy(
                idx3_hbm.at[pl.ds(NPAIR + bases[l], CHUNK)], ig_v[l], sem))
            loads.append(pltpu.make_async_copy(
                idx3_hbm.at[pl.ds(2 * NPAIR + bases[l], CHUNK)],
                ib_v[l].at[pl.ds(0, CHUNK)], sem))
        for cp in loads:
            cp.start()
        for cp in loads:
            cp.wait()
        for l in range(4):
            s2 = SL[l] * SL[l]
            for kk in range(1, 4):
                for j in range(CHUNK // 16):
                    ib_v[l][pl.ds(kk * CHUNK + j * 16, 16)] = (
                        ib_v[l][pl.ds(j * 16, 16)] + kk * s2)
        gats = []
        for l in range(4):
            gats.append(pltpu.make_async_copy(
                wr[l].at[ic_v[l]], vi_v[l], sem))
            gats.append(pltpu.make_async_copy(
                of[l].at[ig_v[l]], vf_v[l], sem))
            gats.append(pltpu.make_async_copy(
                bf[l].at[ib_v[l]], vb_v[l], sem))
        for cp in gats:
            cp.start()
        for cp in gats:
            cp.wait()
        outs = []
        for l in range(4):
            outs.append(pltpu.make_async_copy(vi_v[l], wb_o.at[sls[l]], sem))
            outs.append(pltpu.make_async_copy(
                vf_v[l], pf_o.at[pl.ds(bases[l], CHUNK)], sem))
            for kk in range(4):
                outs.append(pltpu.make_async_copy(
                    vb_v[l].at[pl.ds(kk * CHUNK, CHUNK)],
                    pf_o.at[pl.ds((1 + kk) * NPAIR + bases[l], CHUNK)], sem))
        for cp in outs:
            cp.start()
        for cp in outs:
            cp.wait()

    return k(idx3_f, *writers, *bb_flats, *ob_flats)


def _sig_sp(x):
    e = jnp.exp(-jnp.abs(x))
    inv = 1.0 / (1.0 + e)
    s = jnp.where(x >= 0, inv, e * inv)
    sp = jnp.maximum(x, 0.0) + jnp.log1p(e)
    return s, sp


def _atan_pos(z):
    inv = z > 1.0
    zs = jnp.where(inv, 1.0 / jnp.maximum(z, 1e-30), z)
    x2 = zs * zs
    p = jnp.float32(-0.0117212)
    for c in (0.05265332, -0.11643287, 0.19354346, -0.33262347, 0.99997726):
        p = p * x2 + jnp.float32(c)
    a = zs * p
    return jnp.where(inv, (math.pi / 2) - a, a)


def _finale_body(of0, of1, of2, of3, tt_ref, wb_ref, pf_ref, out_ref, acc):
    i = pl.program_id(0)

    @pl.when(i == 0)
    def _():
        for l in range(4):
            acc[l] = 0.0

    def g0sum(ref):
        x = ref[...]
        s, sp = _sig_sp(x)
        return jnp.sum(sp * s * s)

    acc[0] += g0sum(of0)

    @pl.when(i < 8)
    def _():
        acc[1] += g0sum(of1)

    @pl.when(i < 2)
    def _():
        acc[2] += g0sum(of2)

    @pl.when(i == 0)
    def _():
        acc[3] += g0sum(of3)

    @pl.when(i == 15)
    def _():
        mt = _match(tt_ref)
        valid = mt["valid"]
        validf = valid.astype(jnp.float32)
        gw, gh, aw, ah = mt["gw"], mt["gh"], mt["aw"], mt["ah"]
        fgi = mt["gi"].astype(jnp.float32)
        tbx = mt["gx"] - fgi
        tby = mt["gy"] - fgi

        s0, _ = _sig_sp(pf_ref[1])
        s1, _ = _sig_sp(pf_ref[2])
        s2, _ = _sig_sp(pf_ref[3])
        s3, _ = _sig_sp(pf_ref[4])
        px = s0 * 2.0 - 0.5
        py = s1 * 2.0 - 0.5
        pw = (s2 * 2.0) ** 2 * aw
        ph = (s3 * 2.0) ** 2 * ah

        b1x1 = px - pw * 0.5
        b1y1 = py - ph * 0.5
        b1x2 = px + pw * 0.5
        b1y2 = py + ph * 0.5
        b2x1 = tbx - gw * 0.5
        b2y1 = tby - gh * 0.5
        b2x2 = tbx + gw * 0.5
        b2y2 = tby + gh * 0.5
        inter = (jnp.clip(jnp.minimum(b1x2, b2x2) - jnp.maximum(b1x1, b2x1), 0.0, None)
                 * jnp.clip(jnp.minimum(b1y2, b2y2) - jnp.maximum(b1y1, b2y1), 0.0, None))
        w1 = b1x2 - b1x1
        h1 = b1y2 - b1y1 + EPS
        w2 = b2x2 - b2x1
        h2 = b2y2 - b2y1 + EPS
        union = w1 * h1 + w2 * h2 - inter + EPS
        iou = inter / union
        cw = jnp.maximum(b1x2, b2x2) - jnp.minimum(b1x1, b2x1)
        ch = jnp.maximum(b1y2, b2y2) - jnp.minimum(b1y1, b2y1)
        c2 = cw * cw + ch * ch + EPS
        rho2 = ((b2x1 + b2x2 - b1x1 - b1x2) ** 2
                + (b2y1 + b2y2 - b1y1 - b1y2) ** 2) * 0.25
        v = (4.0 / math.pi ** 2) * (_atan_pos(w2 / (h2 + EPS))
                                    - _atan_pos(w1 / (h1 + EPS))) ** 2
        alpha = v / (1.0 - iou + v + EPS)
        ciou = iou - (rho2 / c2 + v * alpha)

        omi = 1.0 - ciou
        term = jnp.where(valid, omi * jnp.sqrt(jnp.maximum(omi, 0.0)), 0.0)

        rowiota = (lax.broadcasted_iota(jnp.int32, (NROWS, NTP), 0) * NTP
                   + lax.broadcasted_iota(jnp.int32, (NROWS, NTP), 1))
        win = valid & (wb_ref[...] == rowiota)
        tval = jnp.clip(ciou, 0.0, None)
        xo = pf_ref[0]
        sx, spx = _sig_sp(xo)
        bce = spx - xo * tval
        focal = tval * (1.0 - sx) ** 2 + (1.0 - tval) * sx ** 2
        pwf = jnp.where(tval > 0, 1.5, 1.0)
        corrt = jnp.where(win, pwf * bce * focal - spx * sx * sx, 0.0)

        lbox = jnp.float32(0.0)
        lobj = jnp.float32(0.0)
        for l in range(4):
            mask = ((mt["lrow"] == l) & (mt["arow"] < 3)).astype(jnp.float32)
            ls = jnp.sum(term * mask)
            cnt = jnp.sum(validf * mask)
            cr = jnp.sum(corrt * mask)
            lbox += jnp.where(cnt > 0, ls / jnp.maximum(cnt, 1.0), 0.0) * BAL[l]
            lobj += (acc[l] + cr) / N_CELL[l] * BAL[l]

        lane = lax.broadcasted_iota(jnp.int32, (1, 128), 1)
        out_ref[...] = jnp.where(lane == 0, lbox * 0.1,
                                 jnp.where(lane == 1, lobj, 0.0))


def _finale(objs, tt, wb, pf5):
    return pl.pallas_call(
        _finale_body,
        grid=(16,),
        in_specs=[
            pl.BlockSpec((8, 3, 80, 80), lambda i: (i, 0, 0, 0)),
            pl.BlockSpec((16, 3, 40, 40), lambda i: (jnp.minimum(i, 7), 0, 0, 0)),
            pl.BlockSpec((64, 3, 20, 20), lambda i: (jnp.minimum(i, 1), 0, 0, 0)),
            pl.BlockSpec((128, 3, 10, 10), lambda i: (0, 0, 0, 0)),
            pl.BlockSpec((5, NTP), lambda i: (0, 0)),
            pl.BlockSpec((NROWS, NTP), lambda i: (0, 0)),
            pl.BlockSpec((5, NROWS, NTP), lambda i: (0, 0, 0)),
        ],
        out_specs=pl.BlockSpec((1, 128), lambda i: (0, 0)),
        out_shape=jax.ShapeDtypeStruct((1, 128), jnp.float32),
        scratch_shapes=[pltpu.SMEM((4,), jnp.float32)],
    )(*objs, tt, wb, pf5)


def kernel(bbox_pred_0, obj_pred_0, bbox_pred_1, obj_pred_1,
           bbox_pred_2, obj_pred_2, bbox_pred_3, obj_pred_3, targets):
    bbs = (bbox_pred_0, bbox_pred_1, bbox_pred_2, bbox_pred_3)
    obs = (obj_pred_0, obj_pred_1, obj_pred_2, obj_pred_3)
    bb_f = [x.reshape(-1) for x in bbs]
    ob_f = [x.reshape(-1) for x in obs]
    tt = jnp.pad(targets, ((0, NTP - NT), (0, 0))).T

    idx3 = _prep(tt)
    idx3_f = idx3.reshape(-1)
    writers = _scatter(idx3_f)
    wb, pf = _gather(idx3_f, writers, bb_f, ob_f)

    res = _finale(obs, tt, wb.reshape(NROWS, NTP),
                  pf.reshape(5, NROWS, NTP))
    lbox = res[0, 0]
    lobj = res[0, 1]
    return (lbox + lobj, jnp.stack([lbox, lobj]))


# speedup vs baseline: 1.0262x; 1.0262x over previous
import functools
import math

import jax
import jax.numpy as jnp
from jax import lax
from jax.experimental import pallas as pl
from jax.experimental.pallas import tpu as pltpu
from jax.experimental.pallas import tpu_sc as plsc

SL = (80, 40, 20, 10)
BAL = (8.0, 4.0, 1.0, 0.4)
ANCH = (
    ((4.0, 4.0), (8.0, 7.0), (13.0, 11.0)),
    ((19.0, 17.0), (28.0, 24.0), (41.0, 35.0)),
    ((59.0, 52.0), (87.0, 74.0), (122.0, 98.0)),
    ((165.0, 141.0), (234.0, 187.0), (340.0, 280.0)),
)
IMG = 640.0
BSZ = 128
NT = 2000
NTP = 2048
NROWS = 16
NPAIR = NROWS * NTP
NW = 32
CHUNK = (3 * NTP) // NW
N_CELL = tuple(BSZ * 3 * s * s for s in SL)
WPAD = 32768
EPS = 1e-7

_row_l = [r // 4 for r in range(NROWS)]
_row_a = [r % 4 for r in range(NROWS)]
_ROWV = [a < 3 for a in _row_a]
_S = [float(SL[l]) for l in _row_l]
_AW = [ANCH[l][a][0] * SL[l] / IMG if a < 3 else 1.0
       for l, a in zip(_row_l, _row_a)]
_AH = [ANCH[l][a][1] * SL[l] / IMG if a < 3 else 1.0
       for l, a in zip(_row_l, _row_a)]
_AC = [a if a < 3 else 0 for a in _row_a]
_NC = [N_CELL[l] for l in _row_l]


def _row_consts():
    riota = lax.broadcasted_iota(jnp.int32, (NROWS, 1), 0)
    lrow = riota // 4
    arow = riota % 4
    rowv = arow < 3

    def chain(fn, init):
        out = jnp.full((NROWS, 1), init, jnp.float32)
        for l in range(4):
            for a in range(3):
                out = jnp.where((lrow == l) & (arow == a),
                                jnp.float32(fn(l, a)), out)
        return out

    S = chain(lambda l, a: float(SL[l]), 1.0)
    aw = chain(lambda l, a: ANCH[l][a][0] * SL[l] / IMG, 1.0)
    ah = chain(lambda l, a: ANCH[l][a][1] * SL[l] / IMG, 1.0)
    nc = chain(lambda l, a: float(N_CELL[l]), 0.0)
    ac = jnp.where(rowv, arow, 0)
    return dict(S=S, Si=S.astype(jnp.int32), aw=aw, ah=ah,
                nc=nc.astype(jnp.int32), ac=ac, rowv=rowv, lrow=lrow,
                arow=arow)


def _match(tt_ref):
    rc = _row_consts()
    S, Si, aw, ah, ac, rowv = (rc["S"], rc["Si"], rc["aw"], rc["ah"],
                               rc["ac"], rc["rowv"])

    b = tt_ref[0:1, :]
    x = tt_ref[1:2, :]
    y = tt_ref[2:3, :]
    w = tt_ref[3:4, :]
    h = tt_ref[4:5, :]
    gx = x * S
    gy = y * S
    gw = w * S
    gh = h * S
    rw = gw / aw
    rh = gh / ah
    m = jnp.maximum(jnp.maximum(rw, 1.0 / rw), jnp.maximum(rh, 1.0 / rh))
    valid = (m < 3.0) & rowv
    gi = jnp.clip(gx.astype(jnp.int32), 0, Si - 1)
    gj = jnp.clip(gy.astype(jnp.int32), 0, Si - 1)
    bi = b.astype(jnp.int32)
    cell = ((bi * 3 + ac) * Si + gj) * Si + gi
    return dict(valid=valid, cell=cell, gi=gi, gj=gj, bi=bi, ac=ac, Si=Si,
                gx=gx, gy=gy, gw=gw, gh=gh, aw=aw, ah=ah, nc=rc["nc"],
                lrow=rc["lrow"], arow=rc["arow"])


def _prep_body(tt_ref, idx_ref):
    mt = _match(tt_ref)
    valid, cell = mt["valid"], mt["cell"]
    bb0 = ((mt["bi"] * 12 + mt["ac"] * 4) * mt["Si"] + mt["gj"]) * mt["Si"] + mt["gi"]
    uid = (lax.broadcasted_iota(jnp.int32, (NROWS, NTP), 0) * NTP
           + lax.broadcasted_iota(jnp.int32, (NROWS, NTP), 1))
    spread = (uid * 977) & (WPAD - 1)
    idx_ref[0] = jnp.where(valid, cell, mt["nc"] + spread)
    idx_ref[1] = jnp.where(valid, cell, spread)
    idx_ref[2] = jnp.where(valid, bb0, spread)


def _prep(tt):
    return pl.pallas_call(
        _prep_body,
        grid=(1,),
        in_specs=[pl.BlockSpec((5, NTP), lambda i: (0, 0))],
        out_specs=pl.BlockSpec((3, NROWS, NTP), lambda i: (0, 0, 0)),
        out_shape=jax.ShapeDtypeStruct((3, NROWS, NTP), jnp.int32),
    )(tt)


_MESH = plsc.VectorSubcoreMesh(core_axis_name="c", subcore_axis_name="s")


def _scatter(cells_f):
    @functools.partial(
        pl.kernel,
        out_type=[jax.ShapeDtypeStruct((N_CELL[l] + WPAD,), jnp.int32)
                  for l in range(4)],
        mesh=_MESH,
        scratch_types=([pltpu.VMEM((CHUNK,), jnp.int32)] * 4
                       + [pltpu.VMEM((CHUNK,), jnp.int32)] * 4
                       + [pltpu.SemaphoreType.DMA]),
    )
    def k(idx3_hbm, w0, w1, w2, w3, *scr):
        writers = (w0, w1, w2, w3)
        idx_v = scr[0:4]
        val_v = scr[4:8]
        sem = scr[8]
        wid = lax.axis_index("s") * 2 + lax.axis_index("c")
        bases = [l * (4 * NTP) + wid * CHUNK for l in range(4)]
        loads = [pltpu.make_async_copy(
            idx3_hbm.at[pl.ds(bases[l], CHUNK)], idx_v[l], sem)
            for l in range(4)]
        for cp in loads:
            cp.start()
        for l in range(4):
            for j in range(CHUNK // 16):
                val_v[l][pl.ds(j * 16, 16)] = (
                    lax.iota(jnp.int32, 16) + (bases[l] + j * 16))
        for cp in loads:
            cp.wait()
        scats = [pltpu.make_async_copy(
            val_v[l], writers[l].at[idx_v[l]], sem)
            for l in range(4)]
        for cp in scats:
            cp.start()
        for cp in scats:
            cp.wait()

    return k(cells_f)


def _gather(idx3_f, writers, bb_flats, ob_flats):
    outs = [jax.ShapeDtypeStruct((NPAIR,), jnp.int32),
            jax.ShapeDtypeStruct((5 * NPAIR,), jnp.float32)]

    @functools.partial(
        pl.kernel,
        out_type=outs,
        mesh=_MESH,
        scratch_types=([pltpu.VMEM((CHUNK,), jnp.int32)] * 4
                       + [pltpu.VMEM((CHUNK,), jnp.int32)] * 4
                       + [pltpu.VMEM((4 * CHUNK,), jnp.int32)] * 4
                       + [pltpu.VMEM((CHUNK,), jnp.int32)] * 4
                       + [pltpu.VMEM((CHUNK,), jnp.float32)] * 4
                       + [pltpu.VMEM((4 * CHUNK,), jnp.float32)] * 4
                       + [pltpu.SemaphoreType.DMA]),
    )
    def k(idx3_hbm, w0, w1, w2, w3,
          bf0, bf1, bf2, bf3, of0, of1, of2, of3,
          wb_o, pf_o, *scr):
        wr = (w0, w1, w2, w3)
        bf = (bf0, bf1, bf2, bf3)
        of = (of0, of1, of2, of3)
        ic_v = scr[0:4]
        ig_v = scr[4:8]
        ib_v = scr[8:12]
        vi_v = scr[12:16]
        vf_v = scr[16:20]
        vb_v = scr[20:24]
        sem = scr[24]
        wid = lax.axis_index("s") * 2 + lax.axis_index("c")
        bases = [l * (4 * NTP) + wid * CHUNK for l in range(4)]
        sls = [pl.ds(bases[l], CHUNK) for l in range(4)]
        loads = []
        for l in range(4):
            loads.append(pltpu.make_async_copy(
                idx3_hbm.at[pl.ds(bases[l], CHUNK)], ic_v[l], sem))
            loads.append(pltpu.make_async_copy(
                idx3_hbm.at[pl.ds(NPAIR + bases[l], CHUNK)], ig_v[l], sem))
            loads.append(pltpu.make_async_copy(
                idx3_hbm.at[pl.ds(2 * NPAIR + bases[l], CHUNK)],
                ib_v[l].at[pl.ds(0, CHUNK)], sem))
        for cp in loads:
            cp.start()
        for cp in loads:
            cp.wait()
        for l in range(4):
            s2 = SL[l] * SL[l]
            for kk in range(1, 4):
                for j in range(CHUNK // 16):
                    ib_v[l][pl.ds(kk * CHUNK + j * 16, 16)] = (
                        ib_v[l][pl.ds(j * 16, 16)] + kk * s2)
        gats = []
        for l in range(4):
            gats.append(pltpu.make_async_copy(
                wr[l].at[ic_v[l]], vi_v[l], sem))
            gats.append(pltpu.make_async_copy(
                of[l].at[ig_v[l]], vf_v[l], sem))
            gats.append(pltpu.make_async_copy(
                bf[l].at[ib_v[l]], vb_v[l], sem))
        for cp in gats:
            cp.start()
        for cp in gats:
            cp.wait()
        outs = []
        for l in range(4):
            outs.append(pltpu.make_async_copy(vi_v[l], wb_o.at[sls[l]], sem))
            outs.append(pltpu.make_async_copy(
                vf_v[l], pf_o.at[pl.ds(bases[l], CHUNK)], sem))
            for kk in range(4):
                outs.append(pltpu.make_async_copy(
                    vb_v[l].at[pl.ds(kk * CHUNK, CHUNK)],
                    pf_o.at[pl.ds((1 + kk) * NPAIR + bases[l], CHUNK)], sem))
        for cp in outs:
            cp.start()
        for cp in outs:
            cp.wait()

    return k(idx3_f, *writers, *bb_flats, *ob_flats)


def _sig_sp(x):
    e = jnp.exp(-jnp.abs(x))
    inv = 1.0 / (1.0 + e)
    s = jnp.where(x >= 0, inv, e * inv)
    sp = jnp.maximum(x, 0.0) + jnp.log1p(e)
    return s, sp


def _atan_pos(z):
    inv = z > 1.0
    zs = jnp.where(inv, 1.0 / jnp.maximum(z, 1e-30), z)
    x2 = zs * zs
    p = jnp.float32(-0.0117212)
    for c in (0.05265332, -0.11643287, 0.19354346, -0.33262347, 0.99997726):
        p = p * x2 + jnp.float32(c)
    a = zs * p
    return jnp.where(inv, (math.pi / 2) - a, a)


def _finale_body(of0, of1, of2, of3, tt_ref, wb_ref, pf_ref, out_ref, acc):
    i = pl.program_id(0)

    @pl.when(i == 0)
    def _():
        for l in range(4):
            acc[l] = 0.0

    def g0sum(ref):
        x = ref[...]
        s, sp = _sig_sp(x)
        return jnp.sum(sp * s * s)

    acc[0] += g0sum(of0)

    @pl.when(i < 4)
    def _():
        acc[1] += g0sum(of1)

    @pl.when(i == 0)
    def _():
        acc[2] += g0sum(of2)
        acc[3] += g0sum(of3)

    @pl.when(i == 15)
    def _():
        mt = _match(tt_ref)
        valid = mt["valid"]
        validf = valid.astype(jnp.float32)
        gw, gh, aw, ah = mt["gw"], mt["gh"], mt["aw"], mt["ah"]
        fgi = mt["gi"].astype(jnp.float32)
        tbx = mt["gx"] - fgi
        tby = mt["gy"] - fgi

        s0, _ = _sig_sp(pf_ref[1])
        s1, _ = _sig_sp(pf_ref[2])
        s2, _ = _sig_sp(pf_ref[3])
        s3, _ = _sig_sp(pf_ref[4])
        px = s0 * 2.0 - 0.5
        py = s1 * 2.0 - 0.5
        pw = (s2 * 2.0) ** 2 * aw
        ph = (s3 * 2.0) ** 2 * ah

        b1x1 = px - pw * 0.5
        b1y1 = py - ph * 0.5
        b1x2 = px + pw * 0.5
        b1y2 = py + ph * 0.5
        b2x1 = tbx - gw * 0.5
        b2y1 = tby - gh * 0.5
        b2x2 = tbx + gw * 0.5
        b2y2 = tby + gh * 0.5
        inter = (jnp.clip(jnp.minimum(b1x2, b2x2) - jnp.maximum(b1x1, b2x1), 0.0, None)
                 * jnp.clip(jnp.minimum(b1y2, b2y2) - jnp.maximum(b1y1, b2y1), 0.0, None))
        w1 = b1x2 - b1x1
        h1 = b1y2 - b1y1 + EPS
        w2 = b2x2 - b2x1
        h2 = b2y2 - b2y1 + EPS
        union = w1 * h1 + w2 * h2 - inter + EPS
        iou = inter / union
        cw = jnp.maximum(b1x2, b2x2) - jnp.minimum(b1x1, b2x1)
        ch = jnp.maximum(b1y2, b2y2) - jnp.minimum(b1y1, b2y1)
        c2 = cw * cw + ch * ch + EPS
        rho2 = ((b2x1 + b2x2 - b1x1 - b1x2) ** 2
                + (b2y1 + b2y2 - b1y1 - b1y2) ** 2) * 0.25
        v = (4.0 / math.pi ** 2) * (_atan_pos(w2 / (h2 + EPS))
                                    - _atan_pos(w1 / (h1 + EPS))) ** 2
        alpha = v / (1.0 - iou + v + EPS)
        ciou = iou - (rho2 / c2 + v * alpha)

        omi = 1.0 - ciou
        term = jnp.where(valid, omi * jnp.sqrt(jnp.maximum(omi, 0.0)), 0.0)

        rowiota = (lax.broadcasted_iota(jnp.int32, (NROWS, NTP), 0) * NTP
                   + lax.broadcasted_iota(jnp.int32, (NROWS, NTP), 1))
        win = valid & (wb_ref[...] == rowiota)
        tval = jnp.clip(ciou, 0.0, None)
        xo = pf_ref[0]
        sx, spx = _sig_sp(xo)
        bce = spx - xo * tval
        focal = tval * (1.0 - sx) ** 2 + (1.0 - tval) * sx ** 2
        pwf = jnp.where(tval > 0, 1.5, 1.0)
        corrt = jnp.where(win, pwf * bce * focal - spx * sx * sx, 0.0)

        lbox = jnp.float32(0.0)
        lobj = jnp.float32(0.0)
        for l in range(4):
            mask = ((mt["lrow"] == l) & (mt["arow"] < 3)).astype(jnp.float32)
            ls = jnp.sum(term * mask)
            cnt = jnp.sum(validf * mask)
            cr = jnp.sum(corrt * mask)
            lbox += jnp.where(cnt > 0, ls / jnp.maximum(cnt, 1.0), 0.0) * BAL[l]
            lobj += (acc[l] + cr) / N_CELL[l] * BAL[l]

        lane = lax.broadcasted_iota(jnp.int32, (1, 128), 1)
        out_ref[...] = jnp.where(lane == 0, lbox * 0.1,
                                 jnp.where(lane == 1, lobj, 0.0))


def _finale(of2d, tt, wb, pf5):
    blk = 1200
    return pl.pallas_call(
        _finale_body,
        grid=(16,),
        in_specs=[
            pl.BlockSpec((blk, 128), lambda i: (i, 0)),
            pl.BlockSpec((blk, 128), lambda i: (jnp.minimum(i, 3), 0)),
            pl.BlockSpec((blk, 128), lambda i: (0, 0)),
            pl.BlockSpec((300, 128), lambda i: (0, 0)),
            pl.BlockSpec((5, NTP), lambda i: (0, 0)),
            pl.BlockSpec((NROWS, NTP), lambda i: (0, 0)),
            pl.BlockSpec((5, NROWS, NTP), lambda i: (0, 0, 0)),
        ],
        out_specs=pl.BlockSpec((1, 128), lambda i: (0, 0)),
        out_shape=jax.ShapeDtypeStruct((1, 128), jnp.float32),
        scratch_shapes=[pltpu.SMEM((4,), jnp.float32)],
    )(*of2d, tt, wb, pf5)


def kernel(bbox_pred_0, obj_pred_0, bbox_pred_1, obj_pred_1,
           bbox_pred_2, obj_pred_2, bbox_pred_3, obj_pred_3, targets):
    bbs = (bbox_pred_0, bbox_pred_1, bbox_pred_2, bbox_pred_3)
    obs = (obj_pred_0, obj_pred_1, obj_pred_2, obj_pred_3)
    bb_f = [x.reshape(-1) for x in bbs]
    ob_f = [x.reshape(-1) for x in obs]
    tt = jnp.pad(targets, ((0, NTP - NT), (0, 0))).T

    idx3 = _prep(tt)
    idx3_f = idx3.reshape(-1)
    writers = _scatter(idx3_f)
    wb, pf = _gather(idx3_f, writers, bb_f, ob_f)

    of2d = [f.reshape(-1, 128) for f in ob_f]
    res = _finale(of2d, tt, wb.reshape(NROWS, NTP),
                  pf.reshape(5, NROWS, NTP))
    lbox = res[0, 0]
    lobj = res[0, 1]
    return (lbox + lobj, jnp.stack([lbox, lobj]))


# speedup vs baseline: 1.1827x; 1.1525x over previous
import functools
import math

import jax
import jax.numpy as jnp
from jax import lax
from jax.experimental import pallas as pl
from jax.experimental.pallas import tpu as pltpu
from jax.experimental.pallas import tpu_sc as plsc

SL = (80, 40, 20, 10)
HPAD = (80, 40, 24, 16)
BAL = (8.0, 4.0, 1.0, 0.4)
ANCH = (
    ((4.0, 4.0), (8.0, 7.0), (13.0, 11.0)),
    ((19.0, 17.0), (28.0, 24.0), (41.0, 35.0)),
    ((59.0, 52.0), (87.0, 74.0), (122.0, 98.0)),
    ((165.0, 141.0), (234.0, 187.0), (340.0, 280.0)),
)
IMG = 640.0
BSZ = 128
NT = 2000
NTP = 2048
NROWS = 16
NPAIR = NROWS * NTP
NW = 32
CHUNK = (3 * NTP) // NW
N_CELL = tuple(BSZ * 3 * s * s for s in SL)
WPAD = 32768
EPS = 1e-7

_row_l = [r // 4 for r in range(NROWS)]
_row_a = [r % 4 for r in range(NROWS)]
_ROWV = [a < 3 for a in _row_a]
_S = [float(SL[l]) for l in _row_l]
_AW = [ANCH[l][a][0] * SL[l] / IMG if a < 3 else 1.0
       for l, a in zip(_row_l, _row_a)]
_AH = [ANCH[l][a][1] * SL[l] / IMG if a < 3 else 1.0
       for l, a in zip(_row_l, _row_a)]
_AC = [a if a < 3 else 0 for a in _row_a]
_NC = [N_CELL[l] for l in _row_l]


def _row_consts():
    riota = lax.broadcasted_iota(jnp.int32, (NROWS, 1), 0)
    lrow = riota // 4
    arow = riota % 4
    rowv = arow < 3

    def chain(fn, init):
        out = jnp.full((NROWS, 1), init, jnp.float32)
        for l in range(4):
            for a in range(3):
                out = jnp.where((lrow == l) & (arow == a),
                                jnp.float32(fn(l, a)), out)
        return out

    S = chain(lambda l, a: float(SL[l]), 1.0)
    Hp = chain(lambda l, a: float(HPAD[l]), 1.0)
    aw = chain(lambda l, a: ANCH[l][a][0] * SL[l] / IMG, 1.0)
    ah = chain(lambda l, a: ANCH[l][a][1] * SL[l] / IMG, 1.0)
    nc = chain(lambda l, a: float(N_CELL[l]), 0.0)
    ac = jnp.where(rowv, arow, 0)
    return dict(S=S, Si=S.astype(jnp.int32), aw=aw, ah=ah,
                nc=nc.astype(jnp.int32), ac=ac, rowv=rowv, lrow=lrow,
                arow=arow, Hpi=Hp.astype(jnp.int32))


def _match(tt_ref):
    rc = _row_consts()
    S, Si, aw, ah, ac, rowv = (rc["S"], rc["Si"], rc["aw"], rc["ah"],
                               rc["ac"], rc["rowv"])

    b = tt_ref[0:1, :]
    x = tt_ref[1:2, :]
    y = tt_ref[2:3, :]
    w = tt_ref[3:4, :]
    h = tt_ref[4:5, :]
    gx = x * S
    gy = y * S
    gw = w * S
    gh = h * S
    rw = gw / aw
    rh = gh / ah
    m = jnp.maximum(jnp.maximum(rw, 1.0 / rw), jnp.maximum(rh, 1.0 / rh))
    valid = (m < 3.0) & rowv
    gi = jnp.clip(gx.astype(jnp.int32), 0, Si - 1)
    gj = jnp.clip(gy.astype(jnp.int32), 0, Si - 1)
    bi = b.astype(jnp.int32)
    cell = ((bi * 3 + ac) * Si + gj) * Si + gi
    return dict(valid=valid, cell=cell, gi=gi, gj=gj, bi=bi, ac=ac, Si=Si,
                gx=gx, gy=gy, gw=gw, gh=gh, aw=aw, ah=ah, nc=rc["nc"],
                lrow=rc["lrow"], arow=rc["arow"], Hpi=rc["Hpi"])


def _prep_body(tt_ref, idx_ref):
    mt = _match(tt_ref)
    valid, cell = mt["valid"], mt["cell"]
    cellp = ((mt["bi"] * 3 + mt["ac"]) * mt["Hpi"] + mt["gj"]) * 128 + mt["gi"]
    bb0 = ((mt["bi"] * 12 + mt["ac"] * 4) * mt["Hpi"] + mt["gj"]) * 128 + mt["gi"]
    uid = (lax.broadcasted_iota(jnp.int32, (NROWS, NTP), 0) * NTP
           + lax.broadcasted_iota(jnp.int32, (NROWS, NTP), 1))
    spread = (uid * 977) & (WPAD - 1)
    idx_ref[0] = jnp.where(valid, cell, mt["nc"] + spread)
    idx_ref[1] = jnp.where(valid, cellp, spread)
    idx_ref[2] = jnp.where(valid, bb0, spread)


def _prep(tt):
    return pl.pallas_call(
        _prep_body,
        grid=(1,),
        in_specs=[pl.BlockSpec((5, NTP), lambda i: (0, 0))],
        out_specs=pl.BlockSpec((3, NROWS, NTP), lambda i: (0, 0, 0)),
        out_shape=jax.ShapeDtypeStruct((3, NROWS, NTP), jnp.int32),
    )(tt)


_MESH = plsc.VectorSubcoreMesh(core_axis_name="c", subcore_axis_name="s")


def _scatter(cells_f):
    @functools.partial(
        pl.kernel,
        out_type=[jax.ShapeDtypeStruct((N_CELL[l] + WPAD,), jnp.int32)
                  for l in range(4)],
        mesh=_MESH,
        scratch_types=([pltpu.VMEM((CHUNK,), jnp.int32)] * 4
                       + [pltpu.VMEM((CHUNK,), jnp.int32)] * 4
                       + [pltpu.SemaphoreType.DMA]),
    )
    def k(idx3_hbm, w0, w1, w2, w3, *scr):
        writers = (w0, w1, w2, w3)
        idx_v = scr[0:4]
        val_v = scr[4:8]
        sem = scr[8]
        wid = lax.axis_index("s") * 2 + lax.axis_index("c")
        bases = [l * (4 * NTP) + wid * CHUNK for l in range(4)]
        loads = [pltpu.make_async_copy(
            idx3_hbm.at[pl.ds(bases[l], CHUNK)], idx_v[l], sem)
            for l in range(4)]
        for cp in loads:
            cp.start()
        for l in range(4):
            for j in range(CHUNK // 16):
                val_v[l][pl.ds(j * 16, 16)] = (
                    lax.iota(jnp.int32, 16) + (bases[l] + j * 16))
        for cp in loads:
            cp.wait()
        scats = [pltpu.make_async_copy(
            val_v[l], writers[l].at[idx_v[l]], sem)
            for l in range(4)]
        for cp in scats:
            cp.start()
        for cp in scats:
            cp.wait()

    return k(cells_f)


def _gather(idx3_f, writers, bb_flats, ob_flats):
    outs = [jax.ShapeDtypeStruct((NPAIR,), jnp.int32),
            jax.ShapeDtypeStruct((5 * NPAIR,), jnp.float32)]

    @functools.partial(
        pl.kernel,
        out_type=outs,
        mesh=_MESH,
        scratch_types=([pltpu.VMEM((CHUNK,), jnp.int32)] * 4
                       + [pltpu.VMEM((CHUNK,), jnp.int32)] * 4
                       + [pltpu.VMEM((4 * CHUNK,), jnp.int32)] * 4
                       + [pltpu.VMEM((CHUNK,), jnp.int32)] * 4
                       + [pltpu.VMEM((CHUNK,), jnp.float32)] * 4
                       + [pltpu.VMEM((4 * CHUNK,), jnp.float32)] * 4
                       + [pltpu.SemaphoreType.DMA]),
    )
    def k(idx3_hbm, w0, w1, w2, w3,
          bf0, bf1, bf2, bf3, of0, of1, of2, of3,
          wb_o, pf_o, *scr):
        wr = (w0, w1, w2, w3)
        bf = (bf0, bf1, bf2, bf3)
        of = (of0, of1, of2, of3)
        ic_v = scr[0:4]
        ig_v = scr[4:8]
        ib_v = scr[8:12]
        vi_v = scr[12:16]
        vf_v = scr[16:20]
        vb_v = scr[20:24]
        sem = scr[24]
        wid = lax.axis_index("s") * 2 + lax.axis_index("c")
        bases = [l * (4 * NTP) + wid * CHUNK for l in range(4)]
        sls = [pl.ds(bases[l], CHUNK) for l in range(4)]
        loads = []
        for l in range(4):
            loads.append(pltpu.make_async_copy(
                idx3_hbm.at[pl.ds(bases[l], CHUNK)], ic_v[l], sem))
            loads.append(pltpu.make_async_copy(
                idx3_hbm.at[pl.ds(NPAIR + bases[l], CHUNK)], ig_v[l], sem))
            loads.append(pltpu.make_async_copy(
                idx3_hbm.at[pl.ds(2 * NPAIR + bases[l], CHUNK)],
                ib_v[l].at[pl.ds(0, CHUNK)], sem))
        for cp in loads:
            cp.start()
        for cp in loads:
            cp.wait()
        for l in range(4):
            s2 = HPAD[l] * 128
            for kk in range(1, 4):
                for j in range(CHUNK // 16):
                    ib_v[l][pl.ds(kk * CHUNK + j * 16, 16)] = (
                        ib_v[l][pl.ds(j * 16, 16)] + kk * s2)
        gats = []
        for l in range(4):
            gats.append(pltpu.make_async_copy(
                wr[l].at[ic_v[l]], vi_v[l], sem))
            gats.append(pltpu.make_async_copy(
                of[l].at[ig_v[l]], vf_v[l], sem))
            gats.append(pltpu.make_async_copy(
                bf[l].at[ib_v[l]], vb_v[l], sem))
        for cp in gats:
            cp.start()
        for cp in gats:
            cp.wait()
        outs = []
        for l in range(4):
            outs.append(pltpu.make_async_copy(vi_v[l], wb_o.at[sls[l]], sem))
            outs.append(pltpu.make_async_copy(
                vf_v[l], pf_o.at[pl.ds(bases[l], CHUNK)], sem))
            for kk in range(4):
                outs.append(pltpu.make_async_copy(
                    vb_v[l].at[pl.ds(kk * CHUNK, CHUNK)],
                    pf_o.at[pl.ds((1 + kk) * NPAIR + bases[l], CHUNK)], sem))
        for cp in outs:
            cp.start()
        for cp in outs:
            cp.wait()

    return k(idx3_f, *writers, *bb_flats, *ob_flats)


def _sig_sp(x):
    e = jnp.exp(-jnp.abs(x))
    inv = 1.0 / (1.0 + e)
    s = jnp.where(x >= 0, inv, e * inv)
    sp = jnp.maximum(x, 0.0) + jnp.log1p(e)
    return s, sp


def _atan_pos(z):
    inv = z > 1.0
    zs = jnp.where(inv, 1.0 / jnp.maximum(z, 1e-30), z)
    x2 = zs * zs
    p = jnp.float32(-0.0117212)
    for c in (0.05265332, -0.11643287, 0.19354346, -0.33262347, 0.99997726):
        p = p * x2 + jnp.float32(c)
    a = zs * p
    return jnp.where(inv, (math.pi / 2) - a, a)


def _finale_body(of0, of1, of2, of3, tt_ref, wb_ref, pf_ref, out_ref, acc):
    i = pl.program_id(0)

    @pl.when(i == 0)
    def _():
        for l in range(4):
            acc[l] = 0.0

    def g0sum(ref):
        x = ref[...]
        s, sp = _sig_sp(x)
        return jnp.sum(sp * s * s)

    acc[0] += g0sum(of0)

    @pl.when(i < 8)
    def _():
        acc[1] += g0sum(of1)

    @pl.when(i == 0)
    def _():
        acc[2] += g0sum(of2)
        acc[3] += g0sum(of3)

    @pl.when(i == 15)
    def _():
        mt = _match(tt_ref)
        valid = mt["valid"]
        validf = valid.astype(jnp.float32)
        gw, gh, aw, ah = mt["gw"], mt["gh"], mt["aw"], mt["ah"]
        fgi = mt["gi"].astype(jnp.float32)
        tbx = mt["gx"] - fgi
        tby = mt["gy"] - fgi

        s0, _ = _sig_sp(pf_ref[1])
        s1, _ = _sig_sp(pf_ref[2])
        s2, _ = _sig_sp(pf_ref[3])
        s3, _ = _sig_sp(pf_ref[4])
        px = s0 * 2.0 - 0.5
        py = s1 * 2.0 - 0.5
        pw = (s2 * 2.0) ** 2 * aw
        ph = (s3 * 2.0) ** 2 * ah

        b1x1 = px - pw * 0.5
        b1y1 = py - ph * 0.5
        b1x2 = px + pw * 0.5
        b1y2 = py + ph * 0.5
        b2x1 = tbx - gw * 0.5
        b2y1 = tby - gh * 0.5
        b2x2 = tbx + gw * 0.5
        b2y2 = tby + gh * 0.5
        inter = (jnp.clip(jnp.minimum(b1x2, b2x2) - jnp.maximum(b1x1, b2x1), 0.0, None)
                 * jnp.clip(jnp.minimum(b1y2, b2y2) - jnp.maximum(b1y1, b2y1), 0.0, None))
        w1 = b1x2 - b1x1
        h1 = b1y2 - b1y1 + EPS
        w2 = b2x2 - b2x1
        h2 = b2y2 - b2y1 + EPS
        union = w1 * h1 + w2 * h2 - inter + EPS
        iou = inter / union
        cw = jnp.maximum(b1x2, b2x2) - jnp.minimum(b1x1, b2x1)
        ch = jnp.maximum(b1y2, b2y2) - jnp.minimum(b1y1, b2y1)
        c2 = cw * cw + ch * ch + EPS
        rho2 = ((b2x1 + b2x2 - b1x1 - b1x2) ** 2
                + (b2y1 + b2y2 - b1y1 - b1y2) ** 2) * 0.25
        v = (4.0 / math.pi ** 2) * (_atan_pos(w2 / (h2 + EPS))
                                    - _atan_pos(w1 / (h1 + EPS))) ** 2
        alpha = v / (1.0 - iou + v + EPS)
        ciou = iou - (rho2 / c2 + v * alpha)

        omi = 1.0 - ciou
        term = jnp.where(valid, omi * jnp.sqrt(jnp.maximum(omi, 0.0)), 0.0)

        rowiota = (lax.broadcasted_iota(jnp.int32, (NROWS, NTP), 0) * NTP
                   + lax.broadcasted_iota(jnp.int32, (NROWS, NTP), 1))
        win = valid & (wb_ref[...] == rowiota)
        tval = jnp.clip(ciou, 0.0, None)
        xo = pf_ref[0]
        sx, spx = _sig_sp(xo)
        bce = spx - xo * tval
        focal = tval * (1.0 - sx) ** 2 + (1.0 - tval) * sx ** 2
        pwf = jnp.where(tval > 0, 1.5, 1.0)
        corrt = jnp.where(win, pwf * bce * focal - spx * sx * sx, 0.0)

        lbox = jnp.float32(0.0)
        lobj = jnp.float32(0.0)
        for l in range(4):
            mask = ((mt["lrow"] == l) & (mt["arow"] < 3)).astype(jnp.float32)
            ls = jnp.sum(term * mask)
            cnt = jnp.sum(validf * mask)
            cr = jnp.sum(corrt * mask)
            lbox += jnp.where(cnt > 0, ls / jnp.maximum(cnt, 1.0), 0.0) * BAL[l]
            lobj += (acc[l] + cr) / N_CELL[l] * BAL[l]

        lane = lax.broadcasted_iota(jnp.int32, (1, 128), 1)
        out_ref[...] = jnp.where(lane == 0, lbox * 0.1,
                                 jnp.where(lane == 1, lobj, 0.0))


def _finale(of2d, tt, wb, pf5):
    return pl.pallas_call(
        _finale_body,
        grid=(16,),
        in_specs=[
            pl.BlockSpec((1920, 128), lambda i: (i, 0)),
            pl.BlockSpec((1920, 128), lambda i: (jnp.minimum(i, 7), 0)),
            pl.BlockSpec((9216, 128), lambda i: (0, 0)),
            pl.BlockSpec((6144, 128), lambda i: (0, 0)),
            pl.BlockSpec((5, NTP), lambda i: (0, 0)),
            pl.BlockSpec((NROWS, NTP), lambda i: (0, 0)),
            pl.BlockSpec((5, NROWS, NTP), lambda i: (0, 0, 0)),
        ],
        out_specs=pl.BlockSpec((1, 128), lambda i: (0, 0)),
        out_shape=jax.ShapeDtypeStruct((1, 128), jnp.float32),
        scratch_shapes=[pltpu.SMEM((4,), jnp.float32)],
    )(*of2d, tt, wb, pf5)


def kernel(bbox_pred_0, obj_pred_0, bbox_pred_1, obj_pred_1,
           bbox_pred_2, obj_pred_2, bbox_pred_3, obj_pred_3, targets):
    bbs = (bbox_pred_0, bbox_pred_1, bbox_pred_2, bbox_pred_3)
    obs = (obj_pred_0, obj_pred_1, obj_pred_2, obj_pred_3)
    bb_f = [jnp.pad(x, ((0, 0), (0, 0), (0, HPAD[l] - SL[l]),
                        (0, 128 - SL[l]))).reshape(-1)
            for l, x in enumerate(bbs)]
    ob_f = [jnp.pad(x, ((0, 0), (0, 0), (0, HPAD[l] - SL[l]),
                        (0, 128 - SL[l])),
                    constant_values=-1e30).reshape(-1)
            for l, x in enumerate(obs)]
    tt = jnp.pad(targets, ((0, NTP - NT), (0, 0))).T

    idx3 = _prep(tt)
    idx3_f = idx3.reshape(-1)
    writers = _scatter(idx3_f)
    wb, pf = _gather(idx3_f, writers, bb_f, ob_f)

    of2d = [f.reshape(-1, 128) for f in ob_f]
    res = _finale(of2d, tt, wb.reshape(NROWS, NTP),
                  pf.reshape(5, NROWS, NTP))
    lbox = res[0, 0]
    lobj = res[0, 1]
    return (lbox + lobj, jnp.stack([lbox, lobj]))
